# baseline (speedup 1.0000x reference)
import functools
import jax
import jax.numpy as jnp
from jax import lax
from jax.experimental import pallas as pl

S = 32
L = 1024
K = 1000
D = 64
TOPK = 64
NEG_INF = float("-inf")

_HI = jax.lax.Precision.HIGHEST


def _first_argmax(v, iota):
    m = jnp.max(v, axis=-1, keepdims=True)
    return jnp.min(jnp.where(v == m, iota, v.shape[-1]), axis=-1)


def _first_argmin(v, iota):
    m = jnp.min(v, axis=-1, keepdims=True)
    return jnp.min(jnp.where(v == m, iota, v.shape[-1]), axis=-1)


def _kth_largest(x, k):
    lo = jnp.min(x, axis=-1, keepdims=True) - 1.0
    hi = jnp.max(x, axis=-1, keepdims=True) + 1.0

    def body(_, carry):
        lo, hi = carry
        mid = 0.5 * (lo + hi)
        cnt = jnp.sum((x >= mid).astype(jnp.int32), axis=-1, keepdims=True)
        pred = cnt >= k
        return (jnp.where(pred, mid, lo), jnp.where(pred, hi, mid))

    lo, hi = lax.fori_loop(0, 48, body, (lo, hi))
    return lo


def _k1_body(frames_ref, cb_ref, p0_ref, p1_ref, filt0_ref, filt1_ref):
    f = frames_ref[...]
    cb = cb_ref[...]
    fn = jnp.sum(f * f, axis=-1, keepdims=True)
    cn = jnp.sum(cb * cb, axis=-1)[None, :]
    fc = lax.dot_general(f, cb, (((1,), (1,)), ((), ())),
                         preferred_element_type=jnp.float32)
    d2 = fn - 2.0 * fc + cn
    iota_k = lax.broadcasted_iota(jnp.int32, (L, K), 1)
    codes = _first_argmin(d2, iota_k)
    onehot = (codes[:, None] == iota_k).astype(jnp.float32)

    for p_ref, filt_ref in ((p0_ref, filt0_ref), (p1_ref, filt1_ref)):
        logits = jnp.dot(onehot, p_ref[...], precision=_HI,
                         preferred_element_type=jnp.float32)
        kth = _kth_largest(logits, TOPK)
        filt_ref[...] = jnp.where(logits >= kth, logits, NEG_INF)


def _k2_body(u0_ref, u1_ref, filt0_ref, filt1_ref, x0_ref, x1_ref):
    iota_k = lax.broadcasted_iota(jnp.int32, (u0_ref.shape[1], K), 1)
    for u_ref, filt_ref, x_ref in ((u0_ref, filt0_ref, x0_ref),
                                   (u1_ref, filt1_ref, x1_ref)):
        u = u_ref[0]
        g = -jnp.log(-jnp.log(u + 1e-9) + 1e-9)
        v = filt_ref[...] + g
        x_ref[0, 0, :] = _first_argmax(v, iota_k)


def _k3_body(x0_ref, x1_ref, frames_ref, cb_ref, err_ref):
    cb = cb_ref[...]
    iota_k = lax.broadcasted_iota(jnp.int32, (L, K), 1)
    oh0 = (x0_ref[0, 0, :][:, None] == iota_k).astype(jnp.float32)
    oh1 = (x1_ref[0, 0, :][:, None] == iota_k).astype(jnp.float32)
    r0 = jnp.dot(oh0, cb, precision=_HI, preferred_element_type=jnp.float32)
    r1 = jnp.dot(oh1, cb, precision=_HI, preferred_element_type=jnp.float32)
    e = 0.5 * r0 + 0.5 * r1 - frames_ref[...]
    err_ref[0, 0, :] = jnp.broadcast_to(jnp.sum(e * e), (128,))


def _k4_body(err_ref, x0_ref, x1_ref, cb_ref, out_ref):
    err = err_ref[:, 0, 0]
    iota_s = lax.broadcasted_iota(jnp.int32, (1, S), 1)
    best = _first_argmin(err[None, :], iota_s)[0]
    sel = (iota_s == best).astype(jnp.int32)[0][:, None]
    cb = cb_ref[...]
    iota_k = lax.broadcasted_iota(jnp.int32, (L, K), 1)
    for i, x_ref in enumerate((x0_ref, x1_ref)):
        x = jnp.sum(x_ref[:, 0, :] * sel, axis=0)
        oh = (x[:, None] == iota_k).astype(jnp.float32)
        out_ref[i] = jnp.dot(oh, cb, precision=_HI,
                             preferred_element_type=jnp.float32)


@jax.jit
def kernel(mixture, u0, u1, codebook, prior0, prior1):
    frames = mixture.reshape(L, D)

    filt0, filt1 = pl.pallas_call(
        _k1_body,
        out_shape=[jax.ShapeDtypeStruct((L, K), jnp.float32)] * 2,
    )(frames, codebook, prior0, prior1)

    TL = 256
    T = L // TL
    x0, x1 = pl.pallas_call(
        _k2_body,
        grid=(T, S),
        in_specs=[
            pl.BlockSpec((1, TL, K), lambda t, s: (s, t, 0)),
            pl.BlockSpec((1, TL, K), lambda t, s: (s, t, 0)),
            pl.BlockSpec((TL, K), lambda t, s: (t, 0)),
            pl.BlockSpec((TL, K), lambda t, s: (t, 0)),
        ],
        out_specs=[
            pl.BlockSpec((1, 1, TL), lambda t, s: (s, 0, t)),
            pl.BlockSpec((1, 1, TL), lambda t, s: (s, 0, t)),
        ],
        out_shape=[jax.ShapeDtypeStruct((S, 1, L), jnp.int32)] * 2,
    )(u0, u1, filt0, filt1)

    err = pl.pallas_call(
        _k3_body,
        grid=(S,),
        in_specs=[
            pl.BlockSpec((1, 1, L), lambda s: (s, 0, 0)),
            pl.BlockSpec((1, 1, L), lambda s: (s, 0, 0)),
            pl.BlockSpec((L, D), lambda s: (0, 0)),
            pl.BlockSpec((K, D), lambda s: (0, 0)),
        ],
        out_specs=pl.BlockSpec((1, 1, 128), lambda s: (s, 0, 0)),
        out_shape=jax.ShapeDtypeStruct((S, 1, 128), jnp.float32),
    )(x0, x1, frames, codebook)

    out = pl.pallas_call(
        _k4_body,
        out_shape=jax.ShapeDtypeStruct((2, L, D), jnp.float32),
    )(err, x0, x1, codebook)

    return out.reshape(2, L * D)


# speedup vs baseline: 1.3262x; 1.3262x over previous
import functools
import jax
import jax.numpy as jnp
from jax import lax
from jax.experimental import pallas as pl
from jax.experimental.pallas import tpu as pltpu
from jax.experimental.pallas import tpu_sc as plsc

S = 32
L = 1024
K = 1000
D = 64
TOPK = 64
NEG_INF = float("-inf")

_HI = jax.lax.Precision.HIGHEST


def _first_argmax(v, iota):
    m = jnp.max(v, axis=-1, keepdims=True)
    return jnp.min(jnp.where(v == m, iota, v.shape[-1]), axis=-1)


def _first_argmin(v, iota):
    m = jnp.min(v, axis=-1, keepdims=True)
    return jnp.min(jnp.where(v == m, iota, v.shape[-1]), axis=-1)


def _kth_largest(x, k):
    lo = jnp.min(x, axis=-1, keepdims=True) - 1.0
    hi = jnp.max(x, axis=-1, keepdims=True) + 1.0

    def body(_, carry):
        lo, hi = carry
        mid = 0.5 * (lo + hi)
        cnt = jnp.sum((x >= mid).astype(jnp.int32), axis=-1, keepdims=True)
        pred = cnt >= k
        return (jnp.where(pred, mid, lo), jnp.where(pred, hi, mid))

    lo, hi = lax.fori_loop(0, 36, body, (lo, hi))
    return lo


def _k1_body(frames_ref, cb_ref, p0_ref, p1_ref, filt0_ref, filt1_ref):
    f = frames_ref[...]
    cb = cb_ref[...]
    fn = jnp.sum(f * f, axis=-1, keepdims=True)
    cn = jnp.sum(cb * cb, axis=-1)[None, :]
    fc = lax.dot_general(f, cb, (((1,), (1,)), ((), ())),
                         preferred_element_type=jnp.float32)
    d2 = fn - 2.0 * fc + cn
    iota_k = lax.broadcasted_iota(jnp.int32, (L, K), 1)
    codes = _first_argmin(d2, iota_k)
    onehot = (codes[:, None] == iota_k).astype(jnp.float32)

    for p_ref, filt_ref in ((p0_ref, filt0_ref), (p1_ref, filt1_ref)):
        logits = jnp.dot(onehot, p_ref[...], precision=_HI,
                         preferred_element_type=jnp.float32)
        kth = _kth_largest(logits, TOPK)
        filt_ref[...] = jnp.where(logits >= kth, logits, NEG_INF)


def _k2_body(u0_ref, u1_ref, filt0_ref, filt1_ref, x0_ref, x1_ref):
    iota_k = lax.broadcasted_iota(jnp.int32, (u0_ref.shape[1], K), 1)
    for u_ref, filt_ref, x_ref in ((u0_ref, filt0_ref, x0_ref),
                                   (u1_ref, filt1_ref, x1_ref)):
        u = u_ref[0]
        g = -jnp.log(-jnp.log(u + 1e-9) + 1e-9)
        v = filt_ref[...] + g
        x_ref[0, 0, :] = _first_argmax(v, iota_k)


def _sc_err_body(x0_hbm, x1_hbm, cb_hbm, frames_hbm, err_hbm,
                 x0_v, x1_v, a_v, b_v, m_v, out_v, sem):
    s = lax.axis_index("c") * 16 + lax.axis_index("s")
    pltpu.sync_copy(x0_hbm.at[s], x0_v)
    pltpu.sync_copy(x1_hbm.at[s], x1_v)
    acc = jnp.zeros((16,), jnp.float32)
    for j in range(8):
        ca = pltpu.async_copy(cb_hbm.at[x0_v.at[j]], a_v, sem)
        cb_ = pltpu.async_copy(cb_hbm.at[x1_v.at[j]], b_v, sem)
        cm = pltpu.async_copy(frames_hbm.at[pl.ds(j * 128, 128)], m_v, sem)
        ca.wait()
        cb_.wait()
        cm.wait()

        def chunk(r, acc):
            out = acc
            for c in range(4):
                csl = pl.ds(c * 16, 16)
                a = a_v[r, csl]
                b = b_v[r, csl]
                m = m_v[r, csl]
                q = 0.5 * a + 0.5 * b
                out = out + (q * q - 2.0 * q * m)
            return out

        acc = lax.fori_loop(0, 128, chunk, acc)
    out_v[...] = acc
    pltpu.sync_copy(out_v, err_hbm.at[s])


_sc_mesh = plsc.VectorSubcoreMesh(core_axis_name="c", subcore_axis_name="s")

_sc_err = functools.partial(
    pl.kernel,
    mesh=_sc_mesh,
    out_type=jax.ShapeDtypeStruct((S, 16), jnp.float32),
    scratch_types=[
        pltpu.VMEM((8, 128), jnp.int32),
        pltpu.VMEM((8, 128), jnp.int32),
        pltpu.VMEM((128, 128), jnp.float32),
        pltpu.VMEM((128, 128), jnp.float32),
        pltpu.VMEM((128, D), jnp.float32),
        pltpu.VMEM((16,), jnp.float32),
        pltpu.SemaphoreType.DMA,
    ],
)(_sc_err_body)


def _k4_body(err_ref, x0_ref, x1_ref, cb_ref, out_ref):
    err = jnp.sum(err_ref[...], axis=-1)
    iota_s = lax.broadcasted_iota(jnp.int32, (1, S), 1)
    best = _first_argmin(err[None, :], iota_s)[0]
    sel = (iota_s == best).astype(jnp.int32)[0][:, None]
    cb = cb_ref[...]
    iota_k = lax.broadcasted_iota(jnp.int32, (L, K), 1)
    for i, x_ref in enumerate((x0_ref, x1_ref)):
        x = jnp.sum(x_ref[:, 0, :] * sel, axis=0)
        oh = (x[:, None] == iota_k).astype(jnp.float32)
        out_ref[i] = jnp.dot(oh, cb, precision=_HI,
                             preferred_element_type=jnp.float32)


@jax.jit
def kernel(mixture, u0, u1, codebook, prior0, prior1):
    frames = mixture.reshape(L, D)

    filt0, filt1 = pl.pallas_call(
        _k1_body,
        out_shape=[jax.ShapeDtypeStruct((L, K), jnp.float32)] * 2,
    )(frames, codebook, prior0, prior1)


    TL = 256
    T = L // TL
    x0, x1 = pl.pallas_call(
        _k2_body,
        grid=(T, S),
        in_specs=[
            pl.BlockSpec((1, TL, K), lambda t, s: (s, t, 0)),
            pl.BlockSpec((1, TL, K), lambda t, s: (s, t, 0)),
            pl.BlockSpec((TL, K), lambda t, s: (t, 0)),
            pl.BlockSpec((TL, K), lambda t, s: (t, 0)),
        ],
        out_specs=[
            pl.BlockSpec((1, 1, TL), lambda t, s: (s, 0, t)),
            pl.BlockSpec((1, 1, TL), lambda t, s: (s, 0, t)),
        ],
        out_shape=[jax.ShapeDtypeStruct((S, 1, L), jnp.int32)] * 2,
    )(u0, u1, filt0, filt1)

    cb_pad = jnp.pad(codebook, ((0, 0), (0, 128 - D)))
    err = _sc_err(x0.reshape(S, 8, 128), x1.reshape(S, 8, 128),
                  cb_pad, frames)

    out = pl.pallas_call(
        _k4_body,
        out_shape=jax.ShapeDtypeStruct((2, L, D), jnp.float32),
    )(err, x0, x1, codebook)

    return out.reshape(2, L * D)


# speedup vs baseline: 1.3760x; 1.0375x over previous
import functools
import jax
import jax.numpy as jnp
from jax import lax
from jax.experimental import pallas as pl
from jax.experimental.pallas import tpu as pltpu
from jax.experimental.pallas import tpu_sc as plsc

S = 32
L = 1024
K = 1000
D = 64
TOPK = 64
NEG_INF = float("-inf")

_HI = jax.lax.Precision.HIGHEST


def _first_argmax(v, iota):
    m = jnp.max(v, axis=-1, keepdims=True)
    return jnp.min(jnp.where(v == m, iota, v.shape[-1]), axis=-1)


def _first_argmin(v, iota):
    m = jnp.min(v, axis=-1, keepdims=True)
    return jnp.min(jnp.where(v == m, iota, v.shape[-1]), axis=-1)


def _kth_largest(x, k):
    lo = jnp.min(x, axis=-1, keepdims=True) - 1.0
    hi = jnp.max(x, axis=-1, keepdims=True) + 1.0

    def body(_, carry):
        lo, hi = carry
        mid = 0.5 * (lo + hi)
        cnt = jnp.sum((x >= mid).astype(jnp.int32), axis=-1, keepdims=True)
        pred = cnt >= k
        return (jnp.where(pred, mid, lo), jnp.where(pred, hi, mid))

    lo, hi = lax.fori_loop(0, 36, body, (lo, hi))
    return lo


def _k1_body(frames_ref, cb_ref, p0_ref, p1_ref, filt0_ref, filt1_ref):
    f = frames_ref[...]
    cb = cb_ref[...]
    fn = jnp.sum(f * f, axis=-1, keepdims=True)
    cn = jnp.sum(cb * cb, axis=-1)[None, :]
    fc = lax.dot_general(f, cb, (((1,), (1,)), ((), ())),
                         preferred_element_type=jnp.float32)
    d2 = fn - 2.0 * fc + cn
    iota_k = lax.broadcasted_iota(jnp.int32, (L, K), 1)
    codes = _first_argmin(d2, iota_k)
    onehot = (codes[:, None] == iota_k).astype(jnp.float32)

    for p_ref, filt_ref in ((p0_ref, filt0_ref), (p1_ref, filt1_ref)):
        logits = jnp.dot(onehot, p_ref[...], precision=_HI,
                         preferred_element_type=jnp.float32)
        kth = _kth_largest(logits, TOPK)
        filt_ref[...] = jnp.where(logits >= kth, logits, NEG_INF)


def _k2_body(u0_ref, u1_ref, filt0_ref, filt1_ref, x0_ref, x1_ref):
    iota_k = lax.broadcasted_iota(jnp.int32, (u0_ref.shape[1], K), 1)
    for u_ref, filt_ref, x_ref in ((u0_ref, filt0_ref, x0_ref),
                                   (u1_ref, filt1_ref, x1_ref)):
        u = u_ref[0]
        g = -jnp.log(-jnp.log(u + 1e-9) + 1e-9)
        v = filt_ref[...] + g
        x_ref[0, 0, :] = _first_argmax(v, iota_k)


def _sc_err_body(x0_hbm, x1_hbm, cb_hbm, frames_hbm, err_hbm,
                 x0_v, x1_v, a_v, b_v, m_v, out_v, sem):
    s = lax.axis_index("c") * 16 + lax.axis_index("s")
    pltpu.sync_copy(x0_hbm.at[s], x0_v)
    pltpu.sync_copy(x1_hbm.at[s], x1_v)
    acc = jnp.zeros((16,), jnp.float32)
    for j in range(8):
        ca = pltpu.async_copy(cb_hbm.at[x0_v.at[j]], a_v, sem)
        cb_ = pltpu.async_copy(cb_hbm.at[x1_v.at[j]], b_v, sem)
        cm = pltpu.async_copy(frames_hbm.at[pl.ds(j * 128, 128)], m_v, sem)
        ca.wait()
        cb_.wait()
        cm.wait()

        def chunk(r, acc):
            out = acc
            for c in range(4):
                csl = pl.ds(c * 16, 16)
                a = a_v[r, csl]
                b = b_v[r, csl]
                m = m_v[r, csl]
                q = 0.5 * a + 0.5 * b
                out = out + (q * q - 2.0 * q * m)
            return out

        acc = lax.fori_loop(0, 128, chunk, acc)
    out_v[...] = acc
    pltpu.sync_copy(out_v, err_hbm.at[s])


_sc_mesh = plsc.VectorSubcoreMesh(core_axis_name="c", subcore_axis_name="s")

_sc_err = functools.partial(
    pl.kernel,
    mesh=_sc_mesh,
    out_type=jax.ShapeDtypeStruct((S, 16), jnp.float32),
    scratch_types=[
        pltpu.VMEM((8, 128), jnp.int32),
        pltpu.VMEM((8, 128), jnp.int32),
        pltpu.VMEM((128, 128), jnp.float32),
        pltpu.VMEM((128, 128), jnp.float32),
        pltpu.VMEM((128, D), jnp.float32),
        pltpu.VMEM((16,), jnp.float32),
        pltpu.SemaphoreType.DMA,
    ],
)(_sc_err_body)


def _k4_body(err_ref, x0_ref, x1_ref, cb_ref, out_ref):
    err = jnp.sum(err_ref[...], axis=-1)
    iota_s = lax.broadcasted_iota(jnp.int32, (1, S), 1)
    best = _first_argmin(err[None, :], iota_s)[0]
    sel = (iota_s == best).astype(jnp.int32)[0][:, None]
    cb = cb_ref[...]
    iota_k = lax.broadcasted_iota(jnp.int32, (L, K), 1)
    for i, x_ref in enumerate((x0_ref, x1_ref)):
        x = jnp.sum(x_ref[:, 0, :] * sel, axis=0)
        oh = (x[:, None] == iota_k).astype(jnp.float32)
        out_ref[i] = jnp.dot(oh, cb, precision=_HI,
                             preferred_element_type=jnp.float32)


@jax.jit
def kernel(mixture, u0, u1, codebook, prior0, prior1):
    frames = mixture.reshape(L, D)

    filt0, filt1 = pl.pallas_call(
        _k1_body,
        out_shape=[jax.ShapeDtypeStruct((L, K), jnp.float32)] * 2,
    )(frames, codebook, prior0, prior1)


    TL = 512
    T = L // TL
    x0, x1 = pl.pallas_call(
        _k2_body,
        grid=(T, S),
        in_specs=[
            pl.BlockSpec((1, TL, K), lambda t, s: (s, t, 0)),
            pl.BlockSpec((1, TL, K), lambda t, s: (s, t, 0)),
            pl.BlockSpec((TL, K), lambda t, s: (t, 0)),
            pl.BlockSpec((TL, K), lambda t, s: (t, 0)),
        ],
        out_specs=[
            pl.BlockSpec((1, 1, TL), lambda t, s: (s, 0, t)),
            pl.BlockSpec((1, 1, TL), lambda t, s: (s, 0, t)),
        ],
        out_shape=[jax.ShapeDtypeStruct((S, 1, L), jnp.int32)] * 2,
    )(u0, u1, filt0, filt1)

    cb_pad = jnp.pad(codebook, ((0, 0), (0, 128 - D)))
    err = _sc_err(x0.reshape(S, 8, 128), x1.reshape(S, 8, 128),
                  cb_pad, frames)

    out = pl.pallas_call(
        _k4_body,
        out_shape=jax.ShapeDtypeStruct((2, L, D), jnp.float32),
    )(err, x0, x1, codebook)

    return out.reshape(2, L * D)


# speedup vs baseline: 1.3897x; 1.0099x over previous
import functools
import jax
import jax.numpy as jnp
from jax import lax
from jax.experimental import pallas as pl
from jax.experimental.pallas import tpu as pltpu
from jax.experimental.pallas import tpu_sc as plsc

S = 32
L = 1024
K = 1000
D = 64
TOPK = 64
NEG_INF = float("-inf")

_HI = jax.lax.Precision.HIGHEST


def _first_argmax(v, iota):
    m = jnp.max(v, axis=-1, keepdims=True)
    return jnp.min(jnp.where(v == m, iota, v.shape[-1]), axis=-1)


def _first_argmin(v, iota):
    m = jnp.min(v, axis=-1, keepdims=True)
    return jnp.min(jnp.where(v == m, iota, v.shape[-1]), axis=-1)


def _kth_largest(x, k):
    lo = jnp.min(x, axis=-1, keepdims=True) - 1.0
    hi = jnp.max(x, axis=-1, keepdims=True) + 1.0

    def body(_, carry):
        lo, hi = carry
        mid = 0.5 * (lo + hi)
        cnt = jnp.sum((x >= mid).astype(jnp.int32), axis=-1, keepdims=True)
        pred = cnt >= k
        return (jnp.where(pred, mid, lo), jnp.where(pred, hi, mid))

    lo, hi = lax.fori_loop(0, 36, body, (lo, hi))
    return lo


def _k1_body(frames_ref, cb_ref, p0_ref, p1_ref, filt0_ref, filt1_ref):
    f = frames_ref[...]
    cb = cb_ref[...]
    fn = jnp.sum(f * f, axis=-1, keepdims=True)
    cn = jnp.sum(cb * cb, axis=-1)[None, :]
    fc = lax.dot_general(f, cb, (((1,), (1,)), ((), ())),
                         preferred_element_type=jnp.float32)
    d2 = fn - 2.0 * fc + cn
    iota_k = lax.broadcasted_iota(jnp.int32, (L, K), 1)
    codes = _first_argmin(d2, iota_k)
    onehot = (codes[:, None] == iota_k).astype(jnp.float32)

    for p_ref, filt_ref in ((p0_ref, filt0_ref), (p1_ref, filt1_ref)):
        logits = jnp.dot(onehot, p_ref[...], precision=_HI,
                         preferred_element_type=jnp.float32)
        kth = _kth_largest(logits, TOPK)
        filt_ref[...] = jnp.where(logits >= kth, logits, NEG_INF)


def _k2_body(u0_ref, u1_ref, filt0_ref, filt1_ref, x0_ref, x1_ref):
    iota_k = lax.broadcasted_iota(jnp.int32, (u0_ref.shape[1], K), 1)
    for u_ref, filt_ref, x_ref in ((u0_ref, filt0_ref, x0_ref),
                                   (u1_ref, filt1_ref, x1_ref)):
        u = u_ref[0]
        g = -jnp.log(-jnp.log(u + 1e-9) + 1e-9)
        v = filt_ref[...] + g
        x_ref[0, 0, :] = _first_argmax(v, iota_k)


def _sc_err_body(x0_hbm, x1_hbm, cb_hbm, frames_hbm, err_hbm,
                 x0_v, x1_v, a_v, b_v, m_v, out_v, sem):
    s = lax.axis_index("c") * 16 + lax.axis_index("s")
    pltpu.sync_copy(x0_hbm.at[s], x0_v)
    pltpu.sync_copy(x1_hbm.at[s], x1_v)
    acc = jnp.zeros((16,), jnp.float32)
    for j in range(8):
        ca = pltpu.async_copy(cb_hbm.at[x0_v.at[j]], a_v, sem)
        cb_ = pltpu.async_copy(cb_hbm.at[x1_v.at[j]], b_v, sem)
        cm = pltpu.async_copy(frames_hbm.at[pl.ds(j * 128, 128)], m_v, sem)
        ca.wait()
        cb_.wait()
        cm.wait()

        def chunk(r, acc):
            out = acc
            for c in range(4):
                csl = pl.ds(c * 16, 16)
                a = a_v[r, csl]
                b = b_v[r, csl]
                m = m_v[r, csl]
                q = 0.5 * a + 0.5 * b
                out = out + (q * q - 2.0 * q * m)
            return out

        acc = lax.fori_loop(0, 128, chunk, acc)
    out_v[...] = acc
    pltpu.sync_copy(out_v, err_hbm.at[s])


_sc_mesh = plsc.VectorSubcoreMesh(core_axis_name="c", subcore_axis_name="s")

_sc_err = functools.partial(
    pl.kernel,
    mesh=_sc_mesh,
    out_type=jax.ShapeDtypeStruct((S, 16), jnp.float32),
    scratch_types=[
        pltpu.VMEM((8, 128), jnp.int32),
        pltpu.VMEM((8, 128), jnp.int32),
        pltpu.VMEM((128, 128), jnp.float32),
        pltpu.VMEM((128, 128), jnp.float32),
        pltpu.VMEM((128, D), jnp.float32),
        pltpu.VMEM((16,), jnp.float32),
        pltpu.SemaphoreType.DMA,
    ],
)(_sc_err_body)


def _k4_body(err_ref, x0_ref, x1_ref, cb_ref, out_ref):
    err = jnp.sum(err_ref[...], axis=-1)
    iota_s = lax.broadcasted_iota(jnp.int32, (1, S), 1)
    best = _first_argmin(err[None, :], iota_s)[0]
    sel = (iota_s == best).astype(jnp.int32)[0][:, None]
    cb = cb_ref[...]
    iota_k = lax.broadcasted_iota(jnp.int32, (L, K), 1)
    for i, x_ref in enumerate((x0_ref, x1_ref)):
        x = jnp.sum(x_ref[:, 0, :] * sel, axis=0)
        oh = (x[:, None] == iota_k).astype(jnp.float32)
        out_ref[i] = jnp.dot(oh, cb, precision=_HI,
                             preferred_element_type=jnp.float32)


@jax.jit
def kernel(mixture, u0, u1, codebook, prior0, prior1):
    frames = mixture.reshape(L, D)

    filt0, filt1 = pl.pallas_call(
        _k1_body,
        out_shape=[jax.ShapeDtypeStruct((L, K), jnp.float32)] * 2,
    )(frames, codebook, prior0, prior1)


    TL = 1024
    T = L // TL
    x0, x1 = pl.pallas_call(
        _k2_body,
        grid=(T, S),
        in_specs=[
            pl.BlockSpec((1, TL, K), lambda t, s: (s, t, 0)),
            pl.BlockSpec((1, TL, K), lambda t, s: (s, t, 0)),
            pl.BlockSpec((TL, K), lambda t, s: (t, 0)),
            pl.BlockSpec((TL, K), lambda t, s: (t, 0)),
        ],
        out_specs=[
            pl.BlockSpec((1, 1, TL), lambda t, s: (s, 0, t)),
            pl.BlockSpec((1, 1, TL), lambda t, s: (s, 0, t)),
        ],
        out_shape=[jax.ShapeDtypeStruct((S, 1, L), jnp.int32)] * 2,
    )(u0, u1, filt0, filt1)

    cb_pad = jnp.pad(codebook, ((0, 0), (0, 128 - D)))
    err = _sc_err(x0.reshape(S, 8, 128), x1.reshape(S, 8, 128),
                  cb_pad, frames)

    out = pl.pallas_call(
        _k4_body,
        out_shape=jax.ShapeDtypeStruct((2, L, D), jnp.float32),
    )(err, x0, x1, codebook)

    return out.reshape(2, L * D)
